# baseline (speedup 1.0000x reference)
import functools

import jax
import jax.numpy as jnp
from jax import lax
from jax.experimental import pallas as pl
from jax.experimental.pallas import tpu as pltpu
from jax.experimental.pallas import tpu_sc as plsc

N = 10000
E = 320000
H = 128
EMB = 64
NC = 2
NS = 16
NW = NC * NS
EPW = E // NW
C = 80
NCHUNK = EPW // C
NBUF = 3
NSRC = 6
NDST = 8
NPER = 24
NSUP = NCHUNK // NPER
NREM = NCHUNK - NSUP * NPER
RPT = 624
TAIL = N - NS * RPT
CW = 16
ZR = 16

_PREC = lax.Precision.HIGHEST


def _sc_agg_body(with_counts, hs, ha, sa_s, sa_d, as_s, as_d, *rest):
    if with_counts:
        acc_a, acc_s, cnt_a, cnt_s = rest[:4]
        scratch = rest[4:]
    else:
        acc_a, acc_s = rest[:2]
        cnt_a = cnt_s = None
        scratch = rest[2:]
    idx_src = scratch[0:NSRC]
    idx_dst = scratch[NSRC:NSRC + NDST]
    p = NSRC + NDST
    rows = scratch[p:p + NBUF]
    gsems = scratch[p + NBUF:p + 2 * NBUF]
    csems = scratch[p + 2 * NBUF:p + 3 * NBUF]
    p = p + 3 * NBUF
    ssems = scratch[p:p + NSRC]
    dsems = scratch[p + NSRC:p + NSRC + NDST]
    p = p + NSRC + NDST
    if with_counts:
        zbuf, zcnt, ones, sh_acc, sh_cnt = scratch[p:]
    else:
        zbuf, sh_acc = scratch[p:]
        zcnt = ones = sh_cnt = None

    cid = lax.axis_index("c")
    sid = lax.axis_index("s")
    wid = sid * NC + cid
    ebase = wid * EPW
    rbase = sid * RPT

    zv = jnp.zeros((16,), jnp.float32)

    def _zrow(r, c):
        for c0 in range(H // 16):
            zbuf[r, pl.ds(c0 * 16, 16)] = zv
        return c
    lax.fori_loop(0, ZR, _zrow, 0)

    if with_counts:
        ov = jnp.ones((16,), jnp.float32)

        def _zcrow(r, c):
            zcnt[r, :] = zv
            return c
        lax.fori_loop(0, ZR, _zcrow, 0)

        def _orow(r, c):
            ones[r, :] = ov
            return c
        lax.fori_loop(0, C, _orow, 0)

    def do_rel(src_hbm, dst_hbm, table_hbm, acc_hbm, cnt_hbm):
        def _zcopy(z, c):
            pltpu.sync_copy(zbuf, sh_acc.at[pl.ds(rbase + z * ZR, ZR)])
            if cnt_hbm is not None:
                pltpu.sync_copy(zcnt, sh_cnt.at[pl.ds(rbase + z * ZR, ZR)])
            return c
        lax.fori_loop(0, RPT // ZR, _zcopy, 0)

        @pl.when(sid == NS - 1)
        def _():
            pltpu.sync_copy(zbuf.at[pl.ds(0, TAIL)],
                            sh_acc.at[pl.ds(NS * RPT, TAIL)])
            if cnt_hbm is not None:
                pltpu.sync_copy(zcnt.at[pl.ds(0, TAIL)],
                                sh_cnt.at[pl.ds(NS * RPT, TAIL)])
        plsc.subcore_barrier()

        def load_src(chunk, slot):
            off = ebase + chunk * C
            pltpu.async_copy(src_hbm.at[pl.ds(off, C)], idx_src[slot],
                             ssems[slot])

        def load_dst(chunk, slot):
            off = ebase + chunk * C
            pltpu.async_copy(dst_hbm.at[pl.ds(off, C)], idx_dst[slot],
                             dsems[slot])

        def fire(chunk_mod_src, rows_slot):
            pltpu.make_async_copy(
                src_hbm.at[pl.ds(0, C)], idx_src[chunk_mod_src],
                ssems[chunk_mod_src]).wait()
            pltpu.async_copy(table_hbm.at[idx_src[chunk_mod_src]],
                             rows[rows_slot], gsems[rows_slot])

        for k in range(NSRC):
            load_src(k, k)
        for k in range(NSRC):
            load_dst(k, k)
        fire(0, 0)

        def chunk(j, t):
            b = t % NBUF
            s6 = t % NSRC
            d8 = t % NDST
            bf = (t + 1) % NBUF
            sf = (t + 1) % NSRC
            pltpu.make_async_copy(
                table_hbm.at[idx_src[s6]], rows[b], gsems[b]).wait()
            @pl.when(j >= 2)
            def _():
                pltpu.make_async_copy(
                    rows[bf], sh_acc.at[idx_dst[0]], csems[bf]).wait()
            pltpu.make_async_copy(
                dst_hbm.at[pl.ds(0, C)], idx_dst[d8], dsems[d8]).wait()
            pltpu.async_copy(rows[b], sh_acc.at[idx_dst[d8]], csems[b],
                             add=True)
            if cnt_hbm is not None:
                pltpu.sync_copy(ones, sh_cnt.at[idx_dst[d8]], add=True)

            @pl.when(j + 1 < NCHUNK)
            def _():
                fire(sf, bf)

            @pl.when(j + NSRC < NCHUNK)
            def _():
                load_src(j + NSRC, s6)

            @pl.when(j + NSRC < NCHUNK)
            def _():
                load_dst(j + NSRC, (t + NSRC) % NDST)

        def outer(o, c):
            j0 = o * NPER
            for t in range(NPER):
                chunk(j0 + t, t)
            return c
        lax.fori_loop(0, NSUP, outer, 0)
        for j in range(NSUP * NPER, NCHUNK):
            chunk(j, j % NPER)
        for j in (NCHUNK - 2, NCHUNK - 1):
            pltpu.make_async_copy(
                rows[j % NBUF], sh_acc.at[idx_dst[0]],
                csems[j % NBUF]).wait()
        plsc.subcore_barrier()

        pltpu.sync_copy(sh_acc.at[pl.ds(rbase, RPT)],
                        acc_hbm.at[cid, pl.ds(rbase, RPT)])
        if cnt_hbm is not None:
            pltpu.sync_copy(sh_cnt.at[pl.ds(rbase, RPT)],
                            cnt_hbm.at[cid, pl.ds(rbase, RPT)])

        @pl.when(sid == NS - 1)
        def _():
            pltpu.sync_copy(sh_acc.at[pl.ds(NS * RPT, TAIL)],
                            acc_hbm.at[cid, pl.ds(NS * RPT, TAIL)])
            if cnt_hbm is not None:
                pltpu.sync_copy(sh_cnt.at[pl.ds(NS * RPT, TAIL)],
                                cnt_hbm.at[cid, pl.ds(NS * RPT, TAIL)])

    do_rel(sa_s, sa_d, hs, acc_a, cnt_a)
    do_rel(as_s, as_d, ha, acc_s, cnt_s)


@functools.cache
def _make_sc_agg(with_counts):
    mesh = plsc.VectorSubcoreMesh(core_axis_name="c", subcore_axis_name="s")
    out_type = [jax.ShapeDtypeStruct((NC, N, H), jnp.float32),
                jax.ShapeDtypeStruct((NC, N, H), jnp.float32)]
    scratch = (
        [pltpu.VMEM((C,), jnp.int32) for _ in range(NSRC + NDST)]
        + [pltpu.VMEM((C, H), jnp.float32) for _ in range(NBUF)]
        + [pltpu.SemaphoreType.DMA for _ in range(2 * NBUF)]
        + [pltpu.SemaphoreType.DMA for _ in range(NSRC + NDST)]
        + [pltpu.VMEM((ZR, H), jnp.float32)]
    )
    if with_counts:
        out_type += [jax.ShapeDtypeStruct((NC, N, CW), jnp.float32),
                     jax.ShapeDtypeStruct((NC, N, CW), jnp.float32)]
        scratch += [pltpu.VMEM((ZR, CW), jnp.float32),
                    pltpu.VMEM((C, CW), jnp.float32),
                    pltpu.VMEM_SHARED((N, H), jnp.float32),
                    pltpu.VMEM_SHARED((N, CW), jnp.float32)]
    else:
        scratch += [pltpu.VMEM_SHARED((N, H), jnp.float32)]
    return pl.kernel(
        functools.partial(_sc_agg_body, with_counts),
        out_type=tuple(out_type),
        mesh=mesh,
        scratch_types=scratch,
        compiler_params=pltpu.CompilerParams(use_tc_tiling_on_sc=False),
        name=f"sc_edge_agg_cnt{int(with_counts)}",
    )


_BLK = 2000
_GRID = N // _BLK


def _ln_relu(x, gamma, beta):
    x = jnp.maximum(x, 0.0)
    mu = jnp.mean(x, axis=-1, keepdims=True)
    xc = x - mu
    var = jnp.mean(xc * xc, axis=-1, keepdims=True)
    return xc * lax.rsqrt(var + 1e-5) * gamma + beta


def _inproj_body(x_s, x_a, w_s, b_s, w_a, b_a, hs_out, ha_out):
    hs_out[...] = jnp.dot(x_s[...], w_s[...], precision=_PREC,
                          preferred_element_type=jnp.float32) + b_s[...]
    ha_out[...] = jnp.dot(x_a[...], w_a[...], precision=_PREC,
                          preferred_element_type=jnp.float32) + b_a[...]


def _in_proj(x_source, x_agent, w_s, b_s, w_a, b_a):
    rowspec = pl.BlockSpec((_BLK, H), lambda i: (i, 0))
    wspec = pl.BlockSpec((H, H), lambda i: (0, 0))
    bspec = pl.BlockSpec((1, H), lambda i: (0, 0))
    return pl.pallas_call(
        _inproj_body,
        grid=(_GRID,),
        in_specs=[rowspec, rowspec, wspec, bspec, wspec, bspec],
        out_specs=[rowspec, rowspec],
        out_shape=[jax.ShapeDtypeStruct((N, H), jnp.float32)] * 2,
        name="tc_in_proj",
    )(x_source, x_agent, w_s, b_s, w_a, b_a)


def _dense_body(acc_a, cnt_a, acc_s, cnt_s, hs_in, ha_in,
                wn_sa, wr_sa, b_sa, wn_as, wr_as, b_as,
                g_src, bl_src, g_agt, bl_agt, hs_out, ha_out):
    def one(acc, cnt, h_dst, wn, wr, b, gamma, beta):
        s = acc[0] + acc[1]
        c = cnt[0, :, 0] + cnt[1, :, 0]
        mean = s / jnp.maximum(c, 1.0)[:, None]
        pre = (jnp.dot(mean, wn[...], precision=_PREC,
                       preferred_element_type=jnp.float32)
               + jnp.dot(h_dst, wr[...], precision=_PREC,
                         preferred_element_type=jnp.float32) + b[...])
        return _ln_relu(pre, gamma[...], beta[...]) + h_dst

    ha_out[...] = one(acc_a[...], cnt_a[...], ha_in[...],
                      wn_sa, wr_sa, b_sa, g_agt, bl_agt)
    hs_out[...] = one(acc_s[...], cnt_s[...], hs_in[...],
                      wn_as, wr_as, b_as, g_src, bl_src)


def _dense_layer(acc_a, cnt_a, acc_s, cnt_s, hs_in, ha_in,
                 wn_sa, wr_sa, b_sa, wn_as, wr_as, b_as,
                 g_src, bl_src, g_agt, bl_agt):
    accspec = pl.BlockSpec((NC, _BLK, H), lambda i: (0, i, 0))
    cntspec = pl.BlockSpec((NC, _BLK, CW), lambda i: (0, i, 0))
    rowspec = pl.BlockSpec((_BLK, H), lambda i: (i, 0))
    wspec = pl.BlockSpec((H, H), lambda i: (0, 0))
    bspec = pl.BlockSpec((1, H), lambda i: (0, 0))
    return pl.pallas_call(
        _dense_body,
        grid=(_GRID,),
        in_specs=[accspec, cntspec, accspec, cntspec, rowspec, rowspec,
                  wspec, wspec, bspec, wspec, wspec, bspec,
                  bspec, bspec, bspec, bspec],
        out_specs=[rowspec, rowspec],
        out_shape=[jax.ShapeDtypeStruct((N, H), jnp.float32)] * 2,
        name="tc_dense_layer",
    )(acc_a, cnt_a, acc_s, cnt_s, hs_in, ha_in,
      wn_sa, wr_sa, b_sa, wn_as, wr_as, b_as,
      g_src, bl_src, g_agt, bl_agt)


def _outproj_body(hs, ha, w, b, gg, o_s, o_a):
    def one(h):
        v = jnp.dot(h, w[...], precision=_PREC,
                    preferred_element_type=jnp.float32) + b[...]
        nrm = jnp.sqrt(jnp.sum(v * v, axis=-1, keepdims=True))
        return v / jnp.maximum(nrm, 1e-12) * gg[...]

    o_s[...] = one(hs[...])
    o_a[...] = one(ha[...])


def _out_proj(hs, ha, w, b, gg):
    rowspec = pl.BlockSpec((_BLK, H), lambda i: (i, 0))
    ospec = pl.BlockSpec((_BLK, EMB), lambda i: (i, 0))
    return pl.pallas_call(
        _outproj_body,
        grid=(_GRID,),
        in_specs=[rowspec, rowspec,
                  pl.BlockSpec((H, EMB), lambda i: (0, 0)),
                  pl.BlockSpec((1, EMB), lambda i: (0, 0)),
                  pl.BlockSpec((1, EMB), lambda i: (0, 0))],
        out_specs=[ospec, ospec],
        out_shape=[jax.ShapeDtypeStruct((N, EMB), jnp.float32)] * 2,
        name="tc_out_proj",
    )(hs, ha, w, b, gg)


def kernel(x_source, x_agent, edge_index_sa, edge_index_as,
           W_lin_src, b_lin_src, W_lin_agt, b_lin_agt,
           Wn_sa_0, Wr_sa_0, b_sa_0, Wn_as_0, Wr_as_0, b_as_0,
           Wn_sa_1, Wr_sa_1, b_sa_1, Wn_as_1, Wr_as_1, b_as_1,
           ln_g_src, ln_b_src, ln_g_agt, ln_b_agt,
           W_out, b_out, g):
    sa_s = edge_index_sa[0].astype(jnp.int32)
    sa_d = edge_index_sa[1].astype(jnp.int32)
    as_s = edge_index_as[0].astype(jnp.int32)
    as_d = edge_index_as[1].astype(jnp.int32)

    r = lambda v: v.reshape(1, -1)

    hs, ha = _in_proj(x_source, x_agent,
                      W_lin_src, r(b_lin_src), W_lin_agt, r(b_lin_agt))

    acc_a, acc_s, cnt_a, cnt_s = _make_sc_agg(True)(
        hs, ha, sa_s, sa_d, as_s, as_d)
    hs, ha = _dense_layer(acc_a, cnt_a, acc_s, cnt_s, hs, ha,
                          Wn_sa_0, Wr_sa_0, r(b_sa_0),
                          Wn_as_0, Wr_as_0, r(b_as_0),
                          r(ln_g_src), r(ln_b_src), r(ln_g_agt), r(ln_b_agt))

    acc_a, acc_s = _make_sc_agg(False)(hs, ha, sa_s, sa_d, as_s, as_d)
    hs, ha = _dense_layer(acc_a, cnt_a, acc_s, cnt_s, hs, ha,
                          Wn_sa_1, Wr_sa_1, r(b_sa_1),
                          Wn_as_1, Wr_as_1, r(b_as_1),
                          r(ln_g_src), r(ln_b_src), r(ln_g_agt), r(ln_b_agt))

    return _out_proj(hs, ha, W_out, r(b_out), r(g))


# speedup vs baseline: 1.4572x; 1.4572x over previous
import functools

import jax
import jax.numpy as jnp
from jax import lax
from jax.experimental import pallas as pl
from jax.experimental.pallas import tpu as pltpu
from jax.experimental.pallas import tpu_sc as plsc

N = 10000
E = 320000
H = 128
EMB = 64
NC = 2
NS = 16
NW = NC * NS
EPW = E // NW
C = 80
NCHUNK = EPW // C
NBUF = 3
NIDX = 2 * NBUF
NSUP = NCHUNK // NIDX
NREM = NCHUNK - NSUP * NIDX
RPT = 624
TAIL = N - NS * RPT
CW = 16
ZR = 16

_PREC = lax.Precision.HIGHEST


def _sc_agg_body(with_counts, hs, ha, sa_s, sa_d, as_s, as_d, *rest):
    if with_counts:
        acc_a, acc_s, cnt_a, cnt_s = rest[:4]
        scratch = rest[4:]
    else:
        acc_a, acc_s = rest[:2]
        cnt_a = cnt_s = None
        scratch = rest[2:]
    idx_src = scratch[0:NIDX]
    idx_dst = scratch[NIDX:2 * NIDX]
    rows = scratch[2 * NIDX:2 * NIDX + NBUF]
    p = 2 * NIDX + NBUF
    gsems = scratch[p:p + NBUF]
    ssems = scratch[p + NBUF:p + NBUF + NIDX]
    dsems = scratch[p + NBUF + NIDX:p + NBUF + 2 * NIDX]
    p = p + NBUF + 2 * NIDX
    if with_counts:
        zbuf, zcnt, ones, sh_acc, sh_cnt = scratch[p:]
    else:
        zbuf, sh_acc = scratch[p:]
        zcnt = ones = sh_cnt = None

    cid = lax.axis_index("c")
    sid = lax.axis_index("s")
    wid = sid * NC + cid
    ebase = wid * EPW
    rbase = sid * RPT

    zv = jnp.zeros((16,), jnp.float32)

    def _zrow(r, c):
        for c0 in range(H // 16):
            zbuf[r, pl.ds(c0 * 16, 16)] = zv
        return c
    lax.fori_loop(0, ZR, _zrow, 0)

    if with_counts:
        ov = jnp.ones((16,), jnp.float32)

        def _zcrow(r, c):
            zcnt[r, :] = zv
            return c
        lax.fori_loop(0, ZR, _zcrow, 0)

        def _orow(r, c):
            ones[r, :] = ov
            return c
        lax.fori_loop(0, C, _orow, 0)

    def do_rel(src_hbm, dst_hbm, table_hbm, acc_hbm, cnt_hbm):
        def _zcopy(z, c):
            pltpu.sync_copy(zbuf, sh_acc.at[pl.ds(rbase + z * ZR, ZR)])
            if cnt_hbm is not None:
                pltpu.sync_copy(zcnt, sh_cnt.at[pl.ds(rbase + z * ZR, ZR)])
            return c
        lax.fori_loop(0, RPT // ZR, _zcopy, 0)

        @pl.when(sid == NS - 1)
        def _():
            pltpu.sync_copy(zbuf.at[pl.ds(0, TAIL)],
                            sh_acc.at[pl.ds(NS * RPT, TAIL)])
            if cnt_hbm is not None:
                pltpu.sync_copy(zcnt.at[pl.ds(0, TAIL)],
                                sh_cnt.at[pl.ds(NS * RPT, TAIL)])
        plsc.subcore_barrier()

        def load_idx(chunk, b6):
            off = ebase + chunk * C
            pltpu.async_copy(src_hbm.at[pl.ds(off, C)], idx_src[b6], ssems[b6])
            pltpu.async_copy(dst_hbm.at[pl.ds(off, C)], idx_dst[b6], dsems[b6])

        def fire(b3, b6):
            pltpu.make_async_copy(
                src_hbm.at[pl.ds(0, C)], idx_src[b6], ssems[b6]).wait()
            pltpu.async_copy(table_hbm.at[idx_src[b6]], rows[b3], gsems[b3])

        for b6 in range(NIDX):
            load_idx(b6, b6)
        for b3 in range(NBUF):
            fire(b3, b3)

        def chunk(j, t):
            b3 = t % NBUF
            b6 = t % NIDX
            b6n = (t + NBUF) % NIDX
            pltpu.make_async_copy(
                table_hbm.at[idx_src[b6]], rows[b3], gsems[b3]).wait()
            pltpu.make_async_copy(
                dst_hbm.at[pl.ds(0, C)], idx_dst[b6], dsems[b6]).wait()
            pltpu.sync_copy(rows[b3], sh_acc.at[idx_dst[b6]], add=True)
            if cnt_hbm is not None:
                pltpu.sync_copy(ones, sh_cnt.at[idx_dst[b6]], add=True)

            @pl.when(j + NIDX < NCHUNK)
            def _():
                load_idx(j + NIDX, b6)

            @pl.when(j + NBUF < NCHUNK)
            def _():
                fire(b3, b6n)

        def outer(o, c):
            j0 = o * NIDX
            for t in range(NIDX):
                chunk(j0 + t, t)
            return c
        lax.fori_loop(0, NSUP, outer, 0)
        for j in range(NSUP * NIDX, NCHUNK):
            chunk(j, j % NIDX)
        plsc.subcore_barrier()

        pltpu.sync_copy(sh_acc.at[pl.ds(rbase, RPT)],
                        acc_hbm.at[cid, pl.ds(rbase, RPT)])
        if cnt_hbm is not None:
            pltpu.sync_copy(sh_cnt.at[pl.ds(rbase, RPT)],
                            cnt_hbm.at[cid, pl.ds(rbase, RPT)])

        @pl.when(sid == NS - 1)
        def _():
            pltpu.sync_copy(sh_acc.at[pl.ds(NS * RPT, TAIL)],
                            acc_hbm.at[cid, pl.ds(NS * RPT, TAIL)])
            if cnt_hbm is not None:
                pltpu.sync_copy(sh_cnt.at[pl.ds(NS * RPT, TAIL)],
                                cnt_hbm.at[cid, pl.ds(NS * RPT, TAIL)])

    do_rel(sa_s, sa_d, hs, acc_a, cnt_a)
    do_rel(as_s, as_d, ha, acc_s, cnt_s)


@functools.cache
def _make_sc_agg(with_counts):
    mesh = plsc.VectorSubcoreMesh(core_axis_name="c", subcore_axis_name="s")
    out_type = [jax.ShapeDtypeStruct((NC, N, H), jnp.float32),
                jax.ShapeDtypeStruct((NC, N, H), jnp.float32)]
    scratch = (
        [pltpu.VMEM((C,), jnp.int32) for _ in range(2 * NIDX)]
        + [pltpu.VMEM((C, H), jnp.float32) for _ in range(NBUF)]
        + [pltpu.SemaphoreType.DMA for _ in range(NBUF + 2 * NIDX)]
        + [pltpu.VMEM((ZR, H), jnp.float32)]
    )
    if with_counts:
        out_type += [jax.ShapeDtypeStruct((NC, N, CW), jnp.float32),
                     jax.ShapeDtypeStruct((NC, N, CW), jnp.float32)]
        scratch += [pltpu.VMEM((ZR, CW), jnp.float32),
                    pltpu.VMEM((C, CW), jnp.float32),
                    pltpu.VMEM_SHARED((N, H), jnp.float32),
                    pltpu.VMEM_SHARED((N, CW), jnp.float32)]
    else:
        scratch += [pltpu.VMEM_SHARED((N, H), jnp.float32)]
    return pl.kernel(
        functools.partial(_sc_agg_body, with_counts),
        out_type=tuple(out_type),
        mesh=mesh,
        scratch_types=scratch,
        compiler_params=pltpu.CompilerParams(use_tc_tiling_on_sc=False),
        name=f"sc_edge_agg_cnt{int(with_counts)}",
    )


_BLK = 2000
_GRID = N // _BLK


def _ln_relu(x, gamma, beta):
    x = jnp.maximum(x, 0.0)
    mu = jnp.mean(x, axis=-1, keepdims=True)
    xc = x - mu
    var = jnp.mean(xc * xc, axis=-1, keepdims=True)
    return xc * lax.rsqrt(var + 1e-5) * gamma + beta


def _inproj_body(x_s, x_a, w_s, b_s, w_a, b_a, hs_out, ha_out):
    hs_out[...] = jnp.dot(x_s[...], w_s[...], precision=_PREC,
                          preferred_element_type=jnp.float32) + b_s[...]
    ha_out[...] = jnp.dot(x_a[...], w_a[...], precision=_PREC,
                          preferred_element_type=jnp.float32) + b_a[...]


def _in_proj(x_source, x_agent, w_s, b_s, w_a, b_a):
    rowspec = pl.BlockSpec((_BLK, H), lambda i: (i, 0))
    wspec = pl.BlockSpec((H, H), lambda i: (0, 0))
    bspec = pl.BlockSpec((1, H), lambda i: (0, 0))
    return pl.pallas_call(
        _inproj_body,
        grid=(_GRID,),
        in_specs=[rowspec, rowspec, wspec, bspec, wspec, bspec],
        out_specs=[rowspec, rowspec],
        out_shape=[jax.ShapeDtypeStruct((N, H), jnp.float32)] * 2,
        name="tc_in_proj",
    )(x_source, x_agent, w_s, b_s, w_a, b_a)


def _dense_body(final, acc_a, cnt_a, acc_s, cnt_s, hs_in, ha_in,
                wn_sa, wr_sa, b_sa, wn_as, wr_as, b_as,
                g_src, bl_src, g_agt, bl_agt, *rest):
    if final:
        w_out, b_out, gg, hs_out, ha_out = rest
    else:
        hs_out, ha_out = rest

    def one(acc, cnt, h_dst, wn, wr, b, gamma, beta):
        s = acc[0] + acc[1]
        c = cnt[0, :, 0] + cnt[1, :, 0]
        mean = s / jnp.maximum(c, 1.0)[:, None]
        pre = (jnp.dot(mean, wn[...], precision=_PREC,
                       preferred_element_type=jnp.float32)
               + jnp.dot(h_dst, wr[...], precision=_PREC,
                         preferred_element_type=jnp.float32) + b[...])
        h = _ln_relu(pre, gamma[...], beta[...]) + h_dst
        if not final:
            return h
        v = jnp.dot(h, w_out[...], precision=_PREC,
                    preferred_element_type=jnp.float32) + b_out[...]
        nrm = jnp.sqrt(jnp.sum(v * v, axis=-1, keepdims=True))
        return v / jnp.maximum(nrm, 1e-12) * gg[...]

    ha_out[...] = one(acc_a[...], cnt_a[...], ha_in[...],
                      wn_sa, wr_sa, b_sa, g_agt, bl_agt)
    hs_out[...] = one(acc_s[...], cnt_s[...], hs_in[...],
                      wn_as, wr_as, b_as, g_src, bl_src)


def _dense_layer(acc_a, cnt_a, acc_s, cnt_s, hs_in, ha_in,
                 wn_sa, wr_sa, b_sa, wn_as, wr_as, b_as,
                 g_src, bl_src, g_agt, bl_agt, final_args=None):
    final = final_args is not None
    accspec = pl.BlockSpec((NC, _BLK, H), lambda i: (0, i, 0))
    cntspec = pl.BlockSpec((NC, _BLK, CW), lambda i: (0, i, 0))
    rowspec = pl.BlockSpec((_BLK, H), lambda i: (i, 0))
    wspec = pl.BlockSpec((H, H), lambda i: (0, 0))
    bspec = pl.BlockSpec((1, H), lambda i: (0, 0))
    in_specs = [accspec, cntspec, accspec, cntspec, rowspec, rowspec,
                wspec, wspec, bspec, wspec, wspec, bspec,
                bspec, bspec, bspec, bspec]
    args = (acc_a, cnt_a, acc_s, cnt_s, hs_in, ha_in,
            wn_sa, wr_sa, b_sa, wn_as, wr_as, b_as,
            g_src, bl_src, g_agt, bl_agt)
    if final:
        in_specs += [pl.BlockSpec((H, EMB), lambda i: (0, 0)),
                     pl.BlockSpec((1, EMB), lambda i: (0, 0)),
                     pl.BlockSpec((1, EMB), lambda i: (0, 0))]
        args += tuple(final_args)
        outspec = pl.BlockSpec((_BLK, EMB), lambda i: (i, 0))
        out_shape = [jax.ShapeDtypeStruct((N, EMB), jnp.float32)] * 2
    else:
        outspec = rowspec
        out_shape = [jax.ShapeDtypeStruct((N, H), jnp.float32)] * 2
    return pl.pallas_call(
        functools.partial(_dense_body, final),
        grid=(_GRID,),
        in_specs=in_specs,
        out_specs=[outspec, outspec],
        out_shape=out_shape,
        name=f"tc_dense_layer_f{int(final)}",
    )(*args)


def _outproj_body(hs, ha, w, b, gg, o_s, o_a):
    def one(h):
        v = jnp.dot(h, w[...], precision=_PREC,
                    preferred_element_type=jnp.float32) + b[...]
        nrm = jnp.sqrt(jnp.sum(v * v, axis=-1, keepdims=True))
        return v / jnp.maximum(nrm, 1e-12) * gg[...]

    o_s[...] = one(hs[...])
    o_a[...] = one(ha[...])


def _out_proj(hs, ha, w, b, gg):
    rowspec = pl.BlockSpec((_BLK, H), lambda i: (i, 0))
    ospec = pl.BlockSpec((_BLK, EMB), lambda i: (i, 0))
    return pl.pallas_call(
        _outproj_body,
        grid=(_GRID,),
        in_specs=[rowspec, rowspec,
                  pl.BlockSpec((H, EMB), lambda i: (0, 0)),
                  pl.BlockSpec((1, EMB), lambda i: (0, 0)),
                  pl.BlockSpec((1, EMB), lambda i: (0, 0))],
        out_specs=[ospec, ospec],
        out_shape=[jax.ShapeDtypeStruct((N, EMB), jnp.float32)] * 2,
        name="tc_out_proj",
    )(hs, ha, w, b, gg)


def kernel(x_source, x_agent, edge_index_sa, edge_index_as,
           W_lin_src, b_lin_src, W_lin_agt, b_lin_agt,
           Wn_sa_0, Wr_sa_0, b_sa_0, Wn_as_0, Wr_as_0, b_as_0,
           Wn_sa_1, Wr_sa_1, b_sa_1, Wn_as_1, Wr_as_1, b_as_1,
           ln_g_src, ln_b_src, ln_g_agt, ln_b_agt,
           W_out, b_out, g):
    sa_s = edge_index_sa[0].astype(jnp.int32)
    sa_d = edge_index_sa[1].astype(jnp.int32)
    as_s = edge_index_as[0].astype(jnp.int32)
    as_d = edge_index_as[1].astype(jnp.int32)

    r = lambda v: v.reshape(1, -1)

    hs, ha = _in_proj(x_source, x_agent,
                      W_lin_src, r(b_lin_src), W_lin_agt, r(b_lin_agt))

    acc_a, acc_s, cnt_a, cnt_s = _make_sc_agg(True)(
        hs, ha, sa_s, sa_d, as_s, as_d)
    hs, ha = _dense_layer(acc_a, cnt_a, acc_s, cnt_s, hs, ha,
                          Wn_sa_0, Wr_sa_0, r(b_sa_0),
                          Wn_as_0, Wr_as_0, r(b_as_0),
                          r(ln_g_src), r(ln_b_src), r(ln_g_agt), r(ln_b_agt))

    acc_a, acc_s = _make_sc_agg(False)(hs, ha, sa_s, sa_d, as_s, as_d)
    out_agt, out_src = _dense_layer(
        acc_a, cnt_a, acc_s, cnt_s, hs, ha,
        Wn_sa_1, Wr_sa_1, r(b_sa_1),
        Wn_as_1, Wr_as_1, r(b_as_1),
        r(ln_g_src), r(ln_b_src), r(ln_g_agt), r(ln_b_agt),
        final_args=(W_out, r(b_out), r(g)))

    return (out_src, out_agt)


# speedup vs baseline: 1.4946x; 1.0256x over previous
import functools

import jax
import jax.numpy as jnp
from jax import lax
from jax.experimental import pallas as pl
from jax.experimental.pallas import tpu as pltpu
from jax.experimental.pallas import tpu_sc as plsc

N = 10000
E = 320000
H = 128
EMB = 64
NC = 2
NS = 16
NW = NC * NS
EPW = E // NW
C = 80
NCHUNK = EPW // C
NBUF = 3
NIDX = 2 * NBUF
NSUP = NCHUNK // NIDX
NREM = NCHUNK - NSUP * NIDX
RPT = 624
TAIL = N - NS * RPT
CW = 16
ZR = 16

_PREC = lax.Precision.HIGHEST


def _sc_agg_body(with_counts, hs, ha, sa_s, sa_d, as_s, as_d, *rest):
    if with_counts:
        acc_a, acc_s, cnt_a, cnt_s = rest[:4]
        scratch = rest[4:]
    else:
        acc_a, acc_s = rest[:2]
        cnt_a = cnt_s = None
        scratch = rest[2:]
    idx_src = scratch[0:NIDX]
    idx_dst = scratch[NIDX:2 * NIDX]
    rows = scratch[2 * NIDX:2 * NIDX + NBUF]
    p = 2 * NIDX + NBUF
    gsems = scratch[p:p + NBUF]
    ssems = scratch[p + NBUF:p + NBUF + NIDX]
    dsems = scratch[p + NBUF + NIDX:p + NBUF + 2 * NIDX]
    p = p + NBUF + 2 * NIDX
    if with_counts:
        zbuf, zcnt, ones, sh_acc, sh_cnt = scratch[p:]
    else:
        zbuf, sh_acc = scratch[p:]
        zcnt = ones = sh_cnt = None

    cid = lax.axis_index("c")
    sid = lax.axis_index("s")
    wid = sid * NC + cid
    ebase = wid * EPW
    rbase = sid * RPT

    zv = jnp.zeros((16,), jnp.float32)

    def _zrow(r, c):
        for c0 in range(H // 16):
            zbuf[r, pl.ds(c0 * 16, 16)] = zv
        return c
    lax.fori_loop(0, ZR, _zrow, 0)

    if with_counts:
        ov = jnp.ones((16,), jnp.float32)

        def _zcrow(r, c):
            zcnt[r, :] = zv
            return c
        lax.fori_loop(0, ZR, _zcrow, 0)

        def _orow(r, c):
            ones[r, :] = ov
            return c
        lax.fori_loop(0, C, _orow, 0)

    def do_rel(src_hbm, dst_hbm, table_hbm, acc_hbm, cnt_hbm):
        def _zcopy(z, c):
            pltpu.sync_copy(zbuf, sh_acc.at[pl.ds(rbase + z * ZR, ZR)])
            if cnt_hbm is not None:
                pltpu.sync_copy(zcnt, sh_cnt.at[pl.ds(rbase + z * ZR, ZR)])
            return c
        lax.fori_loop(0, RPT // ZR, _zcopy, 0)

        @pl.when(sid == NS - 1)
        def _():
            pltpu.sync_copy(zbuf.at[pl.ds(0, TAIL)],
                            sh_acc.at[pl.ds(NS * RPT, TAIL)])
            if cnt_hbm is not None:
                pltpu.sync_copy(zcnt.at[pl.ds(0, TAIL)],
                                sh_cnt.at[pl.ds(NS * RPT, TAIL)])
        plsc.subcore_barrier()

        def load_idx(chunk, b6):
            off = ebase + chunk * C
            pltpu.async_copy(src_hbm.at[pl.ds(off, C)], idx_src[b6], ssems[b6])
            pltpu.async_copy(dst_hbm.at[pl.ds(off, C)], idx_dst[b6], dsems[b6])

        def fire(b3, b6):
            pltpu.make_async_copy(
                src_hbm.at[pl.ds(0, C)], idx_src[b6], ssems[b6]).wait()
            pltpu.async_copy(table_hbm.at[idx_src[b6]], rows[b3], gsems[b3])

        for b6 in range(NIDX):
            load_idx(b6, b6)
        for b3 in range(NBUF):
            fire(b3, b3)

        def chunk(j, t):
            b3 = t % NBUF
            b6 = t % NIDX
            b6n = (t + NBUF) % NIDX
            pltpu.make_async_copy(
                table_hbm.at[idx_src[b6]], rows[b3], gsems[b3]).wait()
            pltpu.make_async_copy(
                dst_hbm.at[pl.ds(0, C)], idx_dst[b6], dsems[b6]).wait()
            pltpu.sync_copy(rows[b3], sh_acc.at[idx_dst[b6]], add=True)
            if cnt_hbm is not None:
                pltpu.sync_copy(ones, sh_cnt.at[idx_dst[b6]], add=True)

            @pl.when(j + NIDX < NCHUNK)
            def _():
                load_idx(j + NIDX, b6)

            @pl.when(j + NBUF < NCHUNK)
            def _():
                fire(b3, b6n)

        def outer(o, c):
            j0 = o * NIDX
            for t in range(NIDX):
                chunk(j0 + t, t)
            return c
        lax.fori_loop(0, NSUP, outer, 0)
        for j in range(NSUP * NIDX, NCHUNK):
            chunk(j, j % NIDX)
        plsc.subcore_barrier()

        pltpu.sync_copy(sh_acc.at[pl.ds(rbase, RPT)],
                        acc_hbm.at[cid, pl.ds(rbase, RPT)])
        if cnt_hbm is not None:
            pltpu.sync_copy(sh_cnt.at[pl.ds(rbase, RPT)],
                            cnt_hbm.at[cid, pl.ds(rbase, RPT)])

        @pl.when(sid == NS - 1)
        def _():
            pltpu.sync_copy(sh_acc.at[pl.ds(NS * RPT, TAIL)],
                            acc_hbm.at[cid, pl.ds(NS * RPT, TAIL)])
            if cnt_hbm is not None:
                pltpu.sync_copy(sh_cnt.at[pl.ds(NS * RPT, TAIL)],
                                cnt_hbm.at[cid, pl.ds(NS * RPT, TAIL)])

    do_rel(sa_s, sa_d, hs, acc_a, cnt_a)
    do_rel(as_s, as_d, ha, acc_s, cnt_s)


@functools.cache
def _make_sc_agg(with_counts):
    mesh = plsc.VectorSubcoreMesh(core_axis_name="c", subcore_axis_name="s")
    out_type = [jax.ShapeDtypeStruct((NC, N, H), jnp.float32),
                jax.ShapeDtypeStruct((NC, N, H), jnp.float32)]
    scratch = (
        [pltpu.VMEM((C,), jnp.int32) for _ in range(2 * NIDX)]
        + [pltpu.VMEM((C, H), jnp.float32) for _ in range(NBUF)]
        + [pltpu.SemaphoreType.DMA for _ in range(NBUF + 2 * NIDX)]
        + [pltpu.VMEM((ZR, H), jnp.float32)]
    )
    if with_counts:
        out_type += [jax.ShapeDtypeStruct((NC, N, CW), jnp.float32),
                     jax.ShapeDtypeStruct((NC, N, CW), jnp.float32)]
        scratch += [pltpu.VMEM((ZR, CW), jnp.float32),
                    pltpu.VMEM((C, CW), jnp.float32),
                    pltpu.VMEM_SHARED((N, H), jnp.float32),
                    pltpu.VMEM_SHARED((N, CW), jnp.float32)]
    else:
        scratch += [pltpu.VMEM_SHARED((N, H), jnp.float32)]
    return pl.kernel(
        functools.partial(_sc_agg_body, with_counts),
        out_type=tuple(out_type),
        mesh=mesh,
        scratch_types=scratch,
        compiler_params=pltpu.CompilerParams(use_tc_tiling_on_sc=False),
        name=f"sc_edge_agg_cnt{int(with_counts)}",
    )


_BLK = 2000
_GRID = N // _BLK


def _ln_relu(x, gamma, beta):
    x = jnp.maximum(x, 0.0)
    mu = jnp.mean(x, axis=-1, keepdims=True)
    xc = x - mu
    var = jnp.mean(xc * xc, axis=-1, keepdims=True)
    return xc * lax.rsqrt(var + 1e-5) * gamma + beta


def _inproj_body(x_s, x_a, w_s, b_s, w_a, b_a, hs_out, ha_out):
    hs_out[...] = jnp.dot(x_s[...], w_s[...], precision=_PREC,
                          preferred_element_type=jnp.float32) + b_s[...]
    ha_out[...] = jnp.dot(x_a[...], w_a[...], precision=_PREC,
                          preferred_element_type=jnp.float32) + b_a[...]


def _in_proj(x_source, x_agent, w_s, b_s, w_a, b_a):
    rowspec = pl.BlockSpec((_BLK, H), lambda i: (i, 0))
    wspec = pl.BlockSpec((H, H), lambda i: (0, 0))
    bspec = pl.BlockSpec((1, H), lambda i: (0, 0))
    return pl.pallas_call(
        _inproj_body,
        grid=(_GRID,),
        in_specs=[rowspec, rowspec, wspec, bspec, wspec, bspec],
        out_specs=[rowspec, rowspec],
        out_shape=[jax.ShapeDtypeStruct((N, H), jnp.float32)] * 2,
        name="tc_in_proj",
    )(x_source, x_agent, w_s, b_s, w_a, b_a)


def _dense_body(acc_a, cnt_a, acc_s, cnt_s, hs_in, ha_in,
                wn_sa, wr_sa, b_sa, wn_as, wr_as, b_as,
                g_src, bl_src, g_agt, bl_agt, hs_out, ha_out):
    def one(acc, cnt, h_dst, wn, wr, b, gamma, beta):
        s = acc[0] + acc[1]
        c = cnt[0, :, 0] + cnt[1, :, 0]
        mean = s / jnp.maximum(c, 1.0)[:, None]
        pre = (jnp.dot(mean, wn[...], precision=_PREC,
                       preferred_element_type=jnp.float32)
               + jnp.dot(h_dst, wr[...], precision=_PREC,
                         preferred_element_type=jnp.float32) + b[...])
        return _ln_relu(pre, gamma[...], beta[...]) + h_dst

    ha_out[...] = one(acc_a[...], cnt_a[...], ha_in[...],
                      wn_sa, wr_sa, b_sa, g_agt, bl_agt)
    hs_out[...] = one(acc_s[...], cnt_s[...], hs_in[...],
                      wn_as, wr_as, b_as, g_src, bl_src)


def _dense_layer(acc_a, cnt_a, acc_s, cnt_s, hs_in, ha_in,
                 wn_sa, wr_sa, b_sa, wn_as, wr_as, b_as,
                 g_src, bl_src, g_agt, bl_agt):
    accspec = pl.BlockSpec((NC, _BLK, H), lambda i: (0, i, 0))
    cntspec = pl.BlockSpec((NC, _BLK, CW), lambda i: (0, i, 0))
    rowspec = pl.BlockSpec((_BLK, H), lambda i: (i, 0))
    wspec = pl.BlockSpec((H, H), lambda i: (0, 0))
    bspec = pl.BlockSpec((1, H), lambda i: (0, 0))
    return pl.pallas_call(
        _dense_body,
        grid=(_GRID,),
        in_specs=[accspec, cntspec, accspec, cntspec, rowspec, rowspec,
                  wspec, wspec, bspec, wspec, wspec, bspec,
                  bspec, bspec, bspec, bspec],
        out_specs=[rowspec, rowspec],
        out_shape=[jax.ShapeDtypeStruct((N, H), jnp.float32)] * 2,
        name="tc_dense_layer",
    )(acc_a, cnt_a, acc_s, cnt_s, hs_in, ha_in,
      wn_sa, wr_sa, b_sa, wn_as, wr_as, b_as,
      g_src, bl_src, g_agt, bl_agt)


def _outproj_body(hs, ha, w, b, gg, o_s, o_a):
    def one(h):
        v = jnp.dot(h, w[...], precision=_PREC,
                    preferred_element_type=jnp.float32) + b[...]
        nrm = jnp.sqrt(jnp.sum(v * v, axis=-1, keepdims=True))
        return v / jnp.maximum(nrm, 1e-12) * gg[...]

    o_s[...] = one(hs[...])
    o_a[...] = one(ha[...])


def _out_proj(hs, ha, w, b, gg):
    rowspec = pl.BlockSpec((_BLK, H), lambda i: (i, 0))
    ospec = pl.BlockSpec((_BLK, EMB), lambda i: (i, 0))
    return pl.pallas_call(
        _outproj_body,
        grid=(_GRID,),
        in_specs=[rowspec, rowspec,
                  pl.BlockSpec((H, EMB), lambda i: (0, 0)),
                  pl.BlockSpec((1, EMB), lambda i: (0, 0)),
                  pl.BlockSpec((1, EMB), lambda i: (0, 0))],
        out_specs=[ospec, ospec],
        out_shape=[jax.ShapeDtypeStruct((N, EMB), jnp.float32)] * 2,
        name="tc_out_proj",
    )(hs, ha, w, b, gg)


def kernel(x_source, x_agent, edge_index_sa, edge_index_as,
           W_lin_src, b_lin_src, W_lin_agt, b_lin_agt,
           Wn_sa_0, Wr_sa_0, b_sa_0, Wn_as_0, Wr_as_0, b_as_0,
           Wn_sa_1, Wr_sa_1, b_sa_1, Wn_as_1, Wr_as_1, b_as_1,
           ln_g_src, ln_b_src, ln_g_agt, ln_b_agt,
           W_out, b_out, g):
    sa_s = edge_index_sa[0].astype(jnp.int32)
    sa_d = edge_index_sa[1].astype(jnp.int32)
    as_s = edge_index_as[0].astype(jnp.int32)
    as_d = edge_index_as[1].astype(jnp.int32)

    r = lambda v: v.reshape(1, -1)

    hs, ha = _in_proj(x_source, x_agent,
                      W_lin_src, r(b_lin_src), W_lin_agt, r(b_lin_agt))

    acc_a, acc_s, cnt_a, cnt_s = _make_sc_agg(True)(
        hs, ha, sa_s, sa_d, as_s, as_d)
    hs, ha = _dense_layer(acc_a, cnt_a, acc_s, cnt_s, hs, ha,
                          Wn_sa_0, Wr_sa_0, r(b_sa_0),
                          Wn_as_0, Wr_as_0, r(b_as_0),
                          r(ln_g_src), r(ln_b_src), r(ln_g_agt), r(ln_b_agt))

    acc_a, acc_s = _make_sc_agg(False)(hs, ha, sa_s, sa_d, as_s, as_d)
    hs, ha = _dense_layer(acc_a, cnt_a, acc_s, cnt_s, hs, ha,
                          Wn_sa_1, Wr_sa_1, r(b_sa_1),
                          Wn_as_1, Wr_as_1, r(b_as_1),
                          r(ln_g_src), r(ln_b_src), r(ln_g_agt), r(ln_b_agt))

    return _out_proj(hs, ha, W_out, r(b_out), r(g))
